# baseline (speedup 1.0000x reference)
import functools

import jax
import jax.numpy as jnp
from jax import lax
from jax.experimental import pallas as pl
from jax.experimental.pallas import tpu as pltpu
from jax.experimental.pallas import tpu_sc as plsc

_N_NODES = 10000
_D = 128
_E = 320000
_K = 10
_ALPHA = 0.1

_NC, _NS = 2, 16
_NW = _NC * _NS
_CHUNK = 96
_CPP = 35
_NPHASE = 3
_EPP = _CPP * _CHUNK
_EPW = _NPHASE * _EPP
_E_PAD = _NW * _EPW
_N_PAD = 10240
_RPT = _N_PAD // _NS

_mesh = plsc.VectorSubcoreMesh(core_axis_name="c", subcore_axis_name="s")


@functools.partial(
    pl.kernel,
    out_type=jax.ShapeDtypeStruct((_NC, _N_PAD, _D), jnp.float32),
    mesh=_mesh,
    scratch_types=[
        pltpu.VMEM_SHARED((_N_PAD, _D), jnp.float32),
        pltpu.VMEM((_CPP, _CHUNK), jnp.int32),
        pltpu.VMEM((_CPP, _CHUNK), jnp.int32),
        pltpu.VMEM((_EPP,), jnp.float32),
        pltpu.VMEM((_CHUNK, _D), jnp.float32),
        pltpu.VMEM((_CHUNK, _D), jnp.float32),
        pltpu.SemaphoreType.DMA,
        pltpu.SemaphoreType.DMA,
    ],
    compiler_params=pltpu.CompilerParams(needs_layout_passes=False),
)
def _sc_propagate(h_hbm, src_hbm, dst_hbm, w_hbm, z_hbm, out_hbm,
                  acc, srcb, dstb, wb, msg0, msg1, g0, g1):
    cid = lax.axis_index("c")
    sid = lax.axis_index("s")
    wid = sid * _NC + cid

    pltpu.sync_copy(z_hbm, acc.at[pl.ds(sid * _RPT, _RPT)])
    plsc.subcore_barrier()

    def scale(buf, c):
        @pl.loop(0, _CHUNK, unroll=4)
        def _scale(e):
            widx = jnp.full((16,), c * _CHUNK + e, jnp.int32)
            wv = plsc.load_gather(wb, [widx])
            for j in range(_D // 16):
                sl = pl.ds(j * 16, 16)
                buf[e, sl] = buf[e, sl] * wv

    for p in range(_NPHASE):
        base = wid * _NPHASE + p
        pltpu.sync_copy(src_hbm.at[base], srcb)
        pltpu.sync_copy(dst_hbm.at[base], dstb)
        pltpu.sync_copy(w_hbm.at[base], wb)

        pltpu.async_copy(h_hbm.at[srcb.at[0]], msg0, g0)

        @pl.loop(0, _CPP // 2)
        def _pair(i):
            c0 = 2 * i
            d1 = pltpu.async_copy(h_hbm.at[srcb.at[c0 + 1]], msg1, g1)
            pltpu.make_async_copy(h_hbm.at[srcb.at[c0]], msg0, g0).wait()
            scale(msg0, c0)
            pltpu.sync_copy(msg0, acc.at[dstb.at[c0]], add=True)

            pltpu.async_copy(h_hbm.at[srcb.at[c0 + 2]], msg0, g0)

            d1.wait()
            scale(msg1, c0 + 1)
            pltpu.sync_copy(msg1, acc.at[dstb.at[c0 + 1]], add=True)

        ct = _CPP - 1
        pltpu.make_async_copy(h_hbm.at[srcb.at[ct]], msg0, g0).wait()
        scale(msg0, ct)
        pltpu.sync_copy(msg0, acc.at[dstb.at[ct]], add=True)

    plsc.subcore_barrier()
    rows = pl.ds(sid * _RPT, _RPT)
    pltpu.sync_copy(acc.at[rows], out_hbm.at[cid, rows])


def _combine_body(a_ref, b_ref, f_ref, o_ref):
    o_ref[...] = (1.0 - _ALPHA) * (a_ref[...] + b_ref[...]) + _ALPHA * f_ref[...]


_combine = pl.pallas_call(
    _combine_body,
    grid=(_N_PAD // 640,),
    in_specs=[pl.BlockSpec((640, _D), lambda i: (i, 0))] * 3,
    out_specs=pl.BlockSpec((640, _D), lambda i: (i, 0)),
    out_shape=jax.ShapeDtypeStruct((_N_PAD, _D), jnp.float32),
)


@jax.jit
def kernel(feat, edge_index, edge_weight):
    pad = _E_PAD - _E
    src = jnp.concatenate([edge_index[0], jnp.zeros((pad,), jnp.int32)])
    dst = jnp.concatenate([edge_index[1], jnp.zeros((pad,), jnp.int32)])
    w = jnp.concatenate([edge_weight, jnp.zeros((pad,), jnp.float32)])
    src = src.reshape(_NW * _NPHASE, _CPP, _CHUNK)
    dst = dst.reshape(_NW * _NPHASE, _CPP, _CHUNK)
    w = w.reshape(_NW * _NPHASE, _EPP)
    zrows = jnp.zeros((_RPT, _D), jnp.float32)
    feat_pad = jnp.concatenate(
        [feat, jnp.zeros((_N_PAD - _N_NODES, _D), jnp.float32)])
    h = feat_pad
    for _ in range(_K):
        acc = _sc_propagate(h, src, dst, w, zrows)
        h = _combine(acc[0], acc[1], feat_pad)
    return h[:_N_NODES]


# speedup vs baseline: 1.3927x; 1.3927x over previous
import functools

import jax
import jax.numpy as jnp
from jax import lax
from jax.experimental import pallas as pl
from jax.experimental.pallas import tpu as pltpu
from jax.experimental.pallas import tpu_sc as plsc

_N_NODES = 10000
_D = 128
_E = 320000
_K = 10
_ALPHA = 0.1

_NC, _NS = 2, 16
_NW = _NC * _NS
_CHUNK = 80
_CPP = 25
_NPHASE = 5
_EPP = _CPP * _CHUNK
_EPW = _NPHASE * _EPP
_E_PAD = _NW * _EPW
_N_PAD = 10240
_RPT = _N_PAD // _NS

_mesh = plsc.VectorSubcoreMesh(core_axis_name="c", subcore_axis_name="s")


@functools.partial(
    pl.kernel,
    out_type=jax.ShapeDtypeStruct((_NC, _N_PAD, _D), jnp.float32),
    mesh=_mesh,
    scratch_types=[
        pltpu.VMEM_SHARED((_N_PAD, _D), jnp.float32),
        pltpu.VMEM((_CPP, _CHUNK), jnp.int32),
        pltpu.VMEM((_CPP, _CHUNK), jnp.int32),
        pltpu.VMEM((_EPP,), jnp.float32),
        pltpu.VMEM((_CHUNK, _D), jnp.float32),
        pltpu.VMEM((_CHUNK, _D), jnp.float32),
        pltpu.SemaphoreType.DMA,
        pltpu.SemaphoreType.DMA,
    ],
    compiler_params=pltpu.CompilerParams(needs_layout_passes=False),
)
def _sc_propagate(h_hbm, src_hbm, dst_hbm, w_hbm, z_hbm, out_hbm,
                  acc, srcb, dstb, wb, msg0, msg1, g0, g1):
    cid = lax.axis_index("c")
    sid = lax.axis_index("s")
    wid = sid * _NC + cid

    pltpu.sync_copy(z_hbm, acc.at[pl.ds(sid * _RPT, _RPT)])
    plsc.subcore_barrier()

    def scale(buf, c):
        @pl.loop(0, _CHUNK, unroll=8)
        def _scale(e):
            widx = jnp.full((16,), c * _CHUNK + e, jnp.int32)
            wv = plsc.load_gather(wb, [widx])
            for j in range(_D // 16):
                sl = pl.ds(j * 16, 16)
                buf[e, sl] = buf[e, sl] * wv

    for p in range(_NPHASE):
        base = wid * _NPHASE + p
        pltpu.sync_copy(src_hbm.at[base], srcb)
        pltpu.sync_copy(dst_hbm.at[base], dstb)
        pltpu.sync_copy(w_hbm.at[base], wb)

        pltpu.async_copy(h_hbm.at[srcb.at[0]], msg0, g0)

        @pl.loop(0, _CPP // 2)
        def _pair(i):
            c0 = 2 * i
            d1 = pltpu.async_copy(h_hbm.at[srcb.at[c0 + 1]], msg1, g1)
            pltpu.make_async_copy(h_hbm.at[srcb.at[c0]], msg0, g0).wait()
            scale(msg0, c0)
            pltpu.sync_copy(msg0, acc.at[dstb.at[c0]], add=True)

            pltpu.async_copy(h_hbm.at[srcb.at[c0 + 2]], msg0, g0)

            d1.wait()
            scale(msg1, c0 + 1)
            pltpu.sync_copy(msg1, acc.at[dstb.at[c0 + 1]], add=True)

        ct = _CPP - 1
        pltpu.make_async_copy(h_hbm.at[srcb.at[ct]], msg0, g0).wait()
        scale(msg0, ct)
        pltpu.sync_copy(msg0, acc.at[dstb.at[ct]], add=True)

    plsc.subcore_barrier()
    rows = pl.ds(sid * _RPT, _RPT)
    pltpu.sync_copy(acc.at[rows], out_hbm.at[cid, rows])


def _combine_body(a_ref, b_ref, f_ref, o_ref):
    o_ref[...] = (1.0 - _ALPHA) * (a_ref[...] + b_ref[...]) + _ALPHA * f_ref[...]


_combine = pl.pallas_call(
    _combine_body,
    grid=(_N_PAD // 640,),
    in_specs=[pl.BlockSpec((640, _D), lambda i: (i, 0))] * 3,
    out_specs=pl.BlockSpec((640, _D), lambda i: (i, 0)),
    out_shape=jax.ShapeDtypeStruct((_N_PAD, _D), jnp.float32),
)


@jax.jit
def kernel(feat, edge_index, edge_weight):
    pad = _E_PAD - _E
    src = jnp.concatenate([edge_index[0], jnp.zeros((pad,), jnp.int32)])
    dst = jnp.concatenate([edge_index[1], jnp.zeros((pad,), jnp.int32)])
    w = jnp.concatenate([edge_weight, jnp.zeros((pad,), jnp.float32)])
    src = src.reshape(_NW * _NPHASE, _CPP, _CHUNK)
    dst = dst.reshape(_NW * _NPHASE, _CPP, _CHUNK)
    w = w.reshape(_NW * _NPHASE, _EPP)
    zrows = jnp.zeros((_RPT, _D), jnp.float32)
    feat_pad = jnp.concatenate(
        [feat, jnp.zeros((_N_PAD - _N_NODES, _D), jnp.float32)])
    h = feat_pad
    for _ in range(_K):
        acc = _sc_propagate(h, src, dst, w, zrows)
        h = _combine(acc[0], acc[1], feat_pad)
    return h[:_N_NODES]


# speedup vs baseline: 1.4758x; 1.0597x over previous
import functools

import jax
import jax.numpy as jnp
from jax import lax
from jax.experimental import pallas as pl
from jax.experimental.pallas import tpu as pltpu
from jax.experimental.pallas import tpu_sc as plsc

_N_NODES = 10000
_D = 128
_E = 320000
_K = 10
_ALPHA = 0.1

_NC, _NS = 2, 16
_NW = _NC * _NS
_CHUNK = 80
_CPP = 25
_NPHASE = 5
_EPP = _CPP * _CHUNK
_EPW = _NPHASE * _EPP
_E_PAD = _NW * _EPW
_N_PAD = 10240
_RPT = _N_PAD // _NS

_mesh = plsc.VectorSubcoreMesh(core_axis_name="c", subcore_axis_name="s")


@functools.partial(
    pl.kernel,
    out_type=jax.ShapeDtypeStruct((_NC, _N_PAD, _D), jnp.float32),
    mesh=_mesh,
    scratch_types=[
        pltpu.VMEM_SHARED((_N_PAD, _D), jnp.float32),
        pltpu.VMEM((_CPP, _CHUNK), jnp.int32),
        pltpu.VMEM((_CPP, _CHUNK), jnp.int32),
        pltpu.VMEM((_EPP,), jnp.float32),
        pltpu.VMEM((_CPP, _CHUNK), jnp.int32),
        pltpu.VMEM((_CPP, _CHUNK), jnp.int32),
        pltpu.VMEM((_EPP,), jnp.float32),
        pltpu.VMEM((_CHUNK, _D), jnp.float32),
        pltpu.VMEM((_CHUNK, _D), jnp.float32),
        pltpu.SemaphoreType.DMA,
        pltpu.SemaphoreType.DMA,
        pltpu.SemaphoreType.DMA,
    ],
    compiler_params=pltpu.CompilerParams(needs_layout_passes=False),
)
def _sc_propagate(h_hbm, src_hbm, dst_hbm, w_hbm, z_hbm, out_hbm,
                  acc, srcb0, dstb0, wb0, srcb1, dstb1, wb1,
                  msg0, msg1, g0, g1, st):
    cid = lax.axis_index("c")
    sid = lax.axis_index("s")
    wid = sid * _NC + cid
    srcbs, dstbs, wbs = (srcb0, srcb1), (dstb0, dstb1), (wb0, wb1)

    def stage(pp, idx):
        b = wid * _NPHASE + pp
        pltpu.async_copy(src_hbm.at[b], srcbs[idx], st)
        pltpu.async_copy(dst_hbm.at[b], dstbs[idx], st)
        pltpu.async_copy(w_hbm.at[b], wbs[idx], st)

    def wait_stage(pp, idx):
        b = wid * _NPHASE + pp
        pltpu.make_async_copy(src_hbm.at[b], srcbs[idx], st).wait()
        pltpu.make_async_copy(dst_hbm.at[b], dstbs[idx], st).wait()
        pltpu.make_async_copy(w_hbm.at[b], wbs[idx], st).wait()

    stage(0, 0)

    pltpu.sync_copy(z_hbm, acc.at[pl.ds(sid * _RPT, _RPT)])
    plsc.subcore_barrier()

    def scale(buf, c, wb):
        @pl.loop(0, _CHUNK, unroll=4)
        def _scale(e):
            widx = jnp.full((16,), c * _CHUNK + e, jnp.int32)
            wv = plsc.load_gather(wb, [widx])
            for j in range(_D // 16):
                sl = pl.ds(j * 16, 16)
                buf[e, sl] = buf[e, sl] * wv

    for p in range(_NPHASE):
        cur = p % 2
        srcb, dstb, wb = srcbs[cur], dstbs[cur], wbs[cur]
        if p + 1 < _NPHASE:
            stage(p + 1, (p + 1) % 2)
        wait_stage(p, cur)

        pltpu.async_copy(h_hbm.at[srcb.at[0]], msg0, g0)

        @pl.loop(0, _CPP // 2)
        def _pair(i):
            c0 = 2 * i
            d1 = pltpu.async_copy(h_hbm.at[srcb.at[c0 + 1]], msg1, g1)
            pltpu.make_async_copy(h_hbm.at[srcb.at[c0]], msg0, g0).wait()
            scale(msg0, c0, wb)
            pltpu.sync_copy(msg0, acc.at[dstb.at[c0]], add=True)

            pltpu.async_copy(h_hbm.at[srcb.at[c0 + 2]], msg0, g0)

            d1.wait()
            scale(msg1, c0 + 1, wb)
            pltpu.sync_copy(msg1, acc.at[dstb.at[c0 + 1]], add=True)

        ct = _CPP - 1
        pltpu.make_async_copy(h_hbm.at[srcb.at[ct]], msg0, g0).wait()
        scale(msg0, ct, wb)
        pltpu.sync_copy(msg0, acc.at[dstb.at[ct]], add=True)

    plsc.subcore_barrier()
    rows = pl.ds(sid * _RPT, _RPT)
    pltpu.sync_copy(acc.at[rows], out_hbm.at[cid, rows])


def _combine_body(a_ref, b_ref, f_ref, o_ref):
    o_ref[...] = (1.0 - _ALPHA) * (a_ref[...] + b_ref[...]) + _ALPHA * f_ref[...]


_combine = pl.pallas_call(
    _combine_body,
    grid=(_N_PAD // 640,),
    in_specs=[pl.BlockSpec((640, _D), lambda i: (i, 0))] * 3,
    out_specs=pl.BlockSpec((640, _D), lambda i: (i, 0)),
    out_shape=jax.ShapeDtypeStruct((_N_PAD, _D), jnp.float32),
)


@jax.jit
def kernel(feat, edge_index, edge_weight):
    pad = _E_PAD - _E
    src = jnp.concatenate([edge_index[0], jnp.zeros((pad,), jnp.int32)])
    dst = jnp.concatenate([edge_index[1], jnp.zeros((pad,), jnp.int32)])
    w = jnp.concatenate([edge_weight, jnp.zeros((pad,), jnp.float32)])
    src = src.reshape(_NW * _NPHASE, _CPP, _CHUNK)
    dst = dst.reshape(_NW * _NPHASE, _CPP, _CHUNK)
    w = w.reshape(_NW * _NPHASE, _EPP)
    zrows = jnp.zeros((_RPT, _D), jnp.float32)
    feat_pad = jnp.concatenate(
        [feat, jnp.zeros((_N_PAD - _N_NODES, _D), jnp.float32)])
    h = feat_pad
    for _ in range(_K):
        acc = _sc_propagate(h, src, dst, w, zrows)
        h = _combine(acc[0], acc[1], feat_pad)
    return h[:_N_NODES]
